# Initial kernel scaffold; baseline (speedup 1.0000x reference)
import jax
import jax.numpy as jnp
from jax.experimental import pallas as pl


def kernel(x, edge_index, batch, W0, b0, g0, be0, W1, b1, g1, be1, W2, b2, g2, be2, linW, linb):
    raise NotImplementedError("write your pallas kernel here")


# speedup vs baseline: 5.8300x; 5.8300x over previous
import functools

import jax
import jax.numpy as jnp
from jax import lax
from jax.experimental import pallas as pl
from jax.experimental.pallas import tpu as pltpu
from jax.experimental.pallas import tpu_sc as plsc

N = 10000
NP = 10112
NB = NP // 128
E = 320000
NC = 2
NS = 16
NT = NC * NS
CH = 128
KJ = 80
PAIRS = KJ // 2
EP = NT * KJ * CH
RPT = NP // NS
D = 128
DOUT = 64
NG = 64


def _sc_edge_scatter(xws, srcidx, dstidx, zeros, width):
    mesh = plsc.VectorSubcoreMesh(core_axis_name="c", subcore_axis_name="s")

    @functools.partial(
        pl.kernel,
        out_type=jax.ShapeDtypeStruct((NC, NP, width), jnp.float32),
        mesh=mesh,
        scratch_types=[
            pltpu.VMEM((2, CH), jnp.int32),
            pltpu.VMEM((2, CH), jnp.int32),
            pltpu.VMEM((2, CH), jnp.int32),
            pltpu.VMEM((2, CH), jnp.int32),
            pltpu.VMEM((CH, width), jnp.float32),
            pltpu.VMEM((CH, width), jnp.float32),
            pltpu.VMEM_SHARED((NP, width), jnp.float32),
            pltpu.SemaphoreType.DMA,
            pltpu.SemaphoreType.DMA,
            pltpu.SemaphoreType.DMA,
            pltpu.SemaphoreType.DMA,
            pltpu.SemaphoreType.DMA,
            pltpu.SemaphoreType.DMA,
        ],
    )
    def k(xws_hbm, src_hbm, dst_hbm, zero_hbm, out_hbm,
          sr0, sr1, dr0, dr1, rows0, rows1, acc,
          is0, is1, gs0, gs1, ss0, ss1):
        cid = lax.axis_index("c")
        sid = lax.axis_index("s")
        wid = cid * NS + sid
        base = sid * RPT

        def fetch_pair(i, sr, dr, sem):
            pltpu.async_copy(src_hbm.at[wid, pl.ds(2 * i, 2)], sr, sem)
            pltpu.async_copy(dst_hbm.at[wid, pl.ds(2 * i, 2)], dr, sem)

        def fetch_wait(sr, dr, sem):
            pltpu.make_async_copy(src_hbm.at[wid, pl.ds(0, 2)], sr, sem).wait()
            pltpu.make_async_copy(dst_hbm.at[wid, pl.ds(0, 2)], dr, sem).wait()

        def gather_start(sr, r, buf, sem):
            pltpu.async_copy(xws_hbm.at[sr.at[r]], buf, sem)

        def gather_wait(sr, buf, sem):
            pltpu.make_async_copy(xws_hbm.at[sr.at[0]], buf, sem).wait()

        def scatter_start(dr, r, buf, sem):
            pltpu.async_copy(buf, acc.at[dr.at[r]], sem, add=True)

        def scatter_wait(dr, buf, sem):
            pltpu.make_async_copy(buf, acc.at[dr.at[0]], sem).wait()

        fetch_pair(0, sr0, dr0, is0)
        pltpu.sync_copy(zero_hbm.at[pl.ds(base, RPT)], acc.at[pl.ds(base, RPT)])
        plsc.subcore_barrier()
        fetch_wait(sr0, dr0, is0)
        gather_start(sr0, 0, rows0, gs0)

        def pair_body(i, cur, nxt):
            (srX, drX, _), (srY, drY, isY) = cur, nxt
            gather_wait(srX, rows0, gs0)

            @pl.when(i > 0)
            def _():
                scatter_wait(drY, rows1, ss1)

            @pl.when(i + 1 < PAIRS)
            def _():
                fetch_pair(i + 1, srY, drY, isY)

            gather_start(srX, 1, rows1, gs1)
            scatter_start(drX, 0, rows0, ss0)
            gather_wait(srX, rows1, gs1)
            scatter_wait(drX, rows0, ss0)

            @pl.when(i + 1 < PAIRS)
            def _():
                fetch_wait(srY, drY, isY)
                gather_start(srY, 0, rows0, gs0)

            scatter_start(drX, 1, rows1, ss1)

        def body(t, carry):
            pair_body(2 * t, (sr0, dr0, is0), (sr1, dr1, is1))
            pair_body(2 * t + 1, (sr1, dr1, is1), (sr0, dr0, is0))
            return carry

        lax.fori_loop(0, PAIRS // 2, body, 0)
        scatter_wait(dr1, rows1, ss1)
        plsc.subcore_barrier()
        pltpu.sync_copy(acc.at[pl.ds(base, RPT)],
                        out_hbm.at[cid].at[pl.ds(base, RPT)])

    return k(xws, srcidx, dstidx, zeros)


def _sc_degree(dstidx, ones_rows, zerosw):
    mesh = plsc.VectorSubcoreMesh(core_axis_name="c", subcore_axis_name="s")

    def scatter_start(acc, dst_v, j, buf, sem):
        pltpu.async_copy(buf, acc.at[dst_v.at[j]], sem, add=True)

    def scatter_wait(acc, dst_v, buf, sem):
        pltpu.make_async_copy(buf, acc.at[dst_v.at[0]], sem).wait()

    @functools.partial(
        pl.kernel,
        out_type=jax.ShapeDtypeStruct((NC, NP, D), jnp.float32),
        mesh=mesh,
        scratch_types=[
            pltpu.VMEM((1, CH), jnp.int32),
            pltpu.VMEM((1, CH), jnp.int32),
            pltpu.VMEM((CH, D), jnp.float32),
            pltpu.VMEM_SHARED((NP, D), jnp.float32),
            pltpu.SemaphoreType.DMA,
            pltpu.SemaphoreType.DMA,
            pltpu.SemaphoreType.DMA,
            pltpu.SemaphoreType.DMA,
        ],
    )
    def k(dst_hbm, ones_hbm, zero_hbm, out_hbm, r0, r1, ones_v, acc,
          is0, is1, ss0, ss1):
        cid = lax.axis_index("c")
        sid = lax.axis_index("s")
        wid = cid * NS + sid
        base = sid * RPT

        def fetch(j, r, sem):
            pltpu.async_copy(dst_hbm.at[wid, pl.ds(j, 1)], r, sem)

        def fetch_wait(r, sem):
            pltpu.make_async_copy(dst_hbm.at[wid, pl.ds(0, 1)], r, sem).wait()

        def scatter_start(r, sem):
            pltpu.async_copy(ones_v, acc.at[r.at[0]], sem, add=True)

        def scatter_wait(r, sem):
            pltpu.make_async_copy(ones_v, acc.at[r.at[0]], sem).wait()

        fetch(0, r0, is0)
        fetch(1, r1, is1)
        pltpu.sync_copy(ones_hbm, ones_v)
        pltpu.sync_copy(zero_hbm.at[pl.ds(base, RPT)], acc.at[pl.ds(base, RPT)])
        plsc.subcore_barrier()
        fetch_wait(r0, is0)
        scatter_start(r0, ss0)

        def body(t, carry):
            fetch_wait(r1, is1)
            scatter_start(r1, ss1)
            scatter_wait(r0, ss0)
            fetch(2 * t + 2, r0, is0)
            scatter_wait(r1, ss1)
            fetch(2 * t + 3, r1, is1)
            fetch_wait(r0, is0)
            scatter_start(r0, ss0)
            return carry

        lax.fori_loop(0, KJ // 2 - 1, body, 0)
        fetch_wait(r1, is1)
        scatter_start(r1, ss1)
        scatter_wait(r0, ss0)
        scatter_wait(r1, ss1)
        plsc.subcore_barrier()
        pltpu.sync_copy(acc.at[pl.ds(base, RPT)],
                        out_hbm.at[cid].at[pl.ds(base, RPT)])

    return k(dstidx, ones_rows, zerosw)


def _tc_prep(deg, x, W0):
    def body(deg_ref, x_ref, w_ref, dinv_ref, xws_ref):
        d = deg_ref[0] + deg_ref[1]
        dcol = d[:, 0:1] + 1.0
        dinv = lax.rsqrt(dcol)
        dinvb = jnp.broadcast_to(dinv, (128, D))
        xw = jnp.dot(x_ref[...], w_ref[...], preferred_element_type=jnp.float32)
        dinv_ref[...] = dinvb
        xws_ref[...] = xw * dinvb

    return pl.pallas_call(
        body,
        grid=(NB,),
        in_specs=[
            pl.BlockSpec((NC, 128, D), lambda i: (0, i, 0)),
            pl.BlockSpec((128, D), lambda i: (i, 0)),
            pl.BlockSpec((D, D), lambda i: (0, 0)),
        ],
        out_specs=[
            pl.BlockSpec((128, D), lambda i: (i, 0)),
            pl.BlockSpec((128, D), lambda i: (i, 0)),
        ],
        out_shape=[
            jax.ShapeDtypeStruct((NP, D), jnp.float32),
            jax.ShapeDtypeStruct((NP, D), jnp.float32),
        ],
    )(deg, x, W0)


def _tc_post(acc, xws, dinv, b):
    def body(acc_ref, xws_ref, dinv_ref, b_ref, agg_ref, st_ref):
        i = pl.program_id(0)
        s = acc_ref[0] + acc_ref[1] + xws_ref[...]
        agg = dinv_ref[...] * s + b_ref[...]
        agg_ref[...] = agg
        row = i * 128 + lax.broadcasted_iota(jnp.int32, (128, 1), 0)
        aggm = jnp.where(row < N, agg, 0.0)

        @pl.when(i == 0)
        def _():
            st_ref[...] = jnp.zeros_like(st_ref)

        st_ref[0:1, :] += jnp.sum(aggm, axis=0, keepdims=True)
        st_ref[1:2, :] += jnp.sum(aggm * aggm, axis=0, keepdims=True)

    return pl.pallas_call(
        body,
        grid=(NB,),
        in_specs=[
            pl.BlockSpec((NC, 128, D), lambda i: (0, i, 0)),
            pl.BlockSpec((128, D), lambda i: (i, 0)),
            pl.BlockSpec((128, D), lambda i: (i, 0)),
            pl.BlockSpec((1, D), lambda i: (0, 0)),
        ],
        out_specs=[
            pl.BlockSpec((128, D), lambda i: (i, 0)),
            pl.BlockSpec((2, D), lambda i: (0, 0)),
        ],
        out_shape=[
            jax.ShapeDtypeStruct((NP, D), jnp.float32),
            jax.ShapeDtypeStruct((2, D), jnp.float32),
        ],
    )(acc, xws, dinv, b)


def _bn_relu(agg, st, g, be):
    mu = st[0:1, :] * (1.0 / N)
    ex2 = st[1:2, :] * (1.0 / N)
    var = ex2 - mu * mu
    scale = g * lax.rsqrt(var + 1e-5)
    return jnp.maximum((agg - mu) * scale + be, 0.0)


def _tc_mm(agg, stats, g, be, W, dinv):
    def body(agg_ref, st_ref, g_ref, be_ref, w_ref, dinv_ref, xws_ref):
        h = _bn_relu(agg_ref[...], st_ref[...], g_ref[...], be_ref[...])
        xw = jnp.dot(h, w_ref[...], preferred_element_type=jnp.float32)
        xws_ref[...] = xw * dinv_ref[...]

    return pl.pallas_call(
        body,
        grid=(NB,),
        in_specs=[
            pl.BlockSpec((128, D), lambda i: (i, 0)),
            pl.BlockSpec((2, D), lambda i: (0, 0)),
            pl.BlockSpec((1, D), lambda i: (0, 0)),
            pl.BlockSpec((1, D), lambda i: (0, 0)),
            pl.BlockSpec((D, D), lambda i: (0, 0)),
            pl.BlockSpec((128, D), lambda i: (i, 0)),
        ],
        out_specs=pl.BlockSpec((128, D), lambda i: (i, 0)),
        out_shape=jax.ShapeDtypeStruct((NP, D), jnp.float32),
    )(agg, stats, g, be, W, dinv)


def _tc_final(agg, stats, g, be, batchb, linW, linb):
    def body(agg_ref, st_ref, g_ref, be_ref, bb_ref, lw_ref, lb_ref,
             out_ref, sums, cnts):
        i = pl.program_id(0)
        h = _bn_relu(agg_ref[...], st_ref[...], g_ref[...], be_ref[...])
        gid = lax.broadcasted_iota(jnp.int32, (128, 128), 1)
        p = (bb_ref[...] == gid).astype(jnp.float32)
        dn = (((0,), (0,)), ((), ()))
        ps = lax.dot_general(p, h, dn, preferred_element_type=jnp.float32)
        pc = lax.dot_general(p, jnp.ones((128, 1), jnp.float32), dn,
                             preferred_element_type=jnp.float32)

        @pl.when(i == 0)
        def _():
            sums[...] = jnp.zeros_like(sums)
            cnts[...] = jnp.zeros_like(cnts)

        sums[...] += ps
        cnts[...] += pc

        @pl.when(i == NB - 1)
        def _():
            pooled = sums[...] / jnp.maximum(cnts[...], 1.0)
            o = jnp.dot(pooled, lw_ref[...],
                        preferred_element_type=jnp.float32) + lb_ref[...]
            out_ref[...] = o[:NG, :]

    return pl.pallas_call(
        body,
        grid=(NB,),
        in_specs=[
            pl.BlockSpec((128, D), lambda i: (i, 0)),
            pl.BlockSpec((2, D), lambda i: (0, 0)),
            pl.BlockSpec((1, D), lambda i: (0, 0)),
            pl.BlockSpec((1, D), lambda i: (0, 0)),
            pl.BlockSpec((128, 128), lambda i: (i, 0)),
            pl.BlockSpec((D, DOUT), lambda i: (0, 0)),
            pl.BlockSpec((1, DOUT), lambda i: (0, 0)),
        ],
        out_specs=pl.BlockSpec((NG, DOUT), lambda i: (0, 0)),
        out_shape=jax.ShapeDtypeStruct((NG, DOUT), jnp.float32),
        scratch_shapes=[
            pltpu.VMEM((128, 128), jnp.float32),
            pltpu.VMEM((128, 1), jnp.float32),
        ],
    )(agg, stats, g, be, batchb, linW, linb)


def kernel(x, edge_index, batch, W0, b0, g0, be0, W1, b1, g1, be1,
           W2, b2, g2, be2, linW, linb):
    f32 = jnp.float32
    xp = jnp.zeros((NP, D), f32).at[:N].set(x)
    pad = jnp.full((EP - E,), N, jnp.int32)
    srcp = jnp.concatenate([edge_index[0].astype(jnp.int32), pad]
                           ).reshape(NT, KJ, CH)
    dstp = jnp.concatenate([edge_index[1].astype(jnp.int32), pad]
                           ).reshape(NT, KJ, CH)
    batchp = jnp.concatenate([batch.astype(jnp.int32),
                              jnp.full((NP - N,), NG, jnp.int32)])
    batchb = jnp.broadcast_to(batchp[:, None], (NP, 128))
    zerosD = jnp.zeros((NP, D), f32)
    ones128 = jnp.ones((CH, D), f32)

    deg = _sc_degree(dstp, ones128, zerosD)
    dinv, xws = _tc_prep(deg, xp, W0)

    layers = ((b0, g0, be0, W1), (b1, g1, be1, W2), (b2, g2, be2, None))
    agg = stats = None
    for b, g, be, Wnext in layers:
        acc = _sc_edge_scatter(xws, srcp, dstp, zerosD, D)
        agg, stats = _tc_post(acc, xws, dinv, b.reshape(1, D))
        if Wnext is not None:
            xws = _tc_mm(agg, stats, g.reshape(1, D), be.reshape(1, D),
                         Wnext, dinv)

    return _tc_final(agg, stats, g2.reshape(1, D), be2.reshape(1, D),
                     batchb, linW, linb.reshape(1, DOUT))


# speedup vs baseline: 7.6135x; 1.3059x over previous
import functools

import jax
import jax.numpy as jnp
from jax import lax
from jax.experimental import pallas as pl
from jax.experimental.pallas import tpu as pltpu
from jax.experimental.pallas import tpu_sc as plsc

N = 10000
NP = 10112
NB = NP // 128
E = 320000
NC = 2
NS = 16
NT = NC * NS
CH = 128
KJ = 80
PAIRS = KJ // 2
EP = NT * KJ * CH
TCH = EP // CH
KJ0 = 132
KJ1 = 28
RPT = NP // NS
D = 128
DOUT = 64
NG = 64


def _sc_edge_scatter(xws, srcidx, dstidx, zeros, width):
    mesh = plsc.VectorSubcoreMesh(core_axis_name="c", subcore_axis_name="s")

    @functools.partial(
        pl.kernel,
        out_type=jax.ShapeDtypeStruct((NC, NP, width), jnp.float32),
        mesh=mesh,
        scratch_types=[
            pltpu.VMEM((2, CH), jnp.int32),
            pltpu.VMEM((2, CH), jnp.int32),
            pltpu.VMEM((2, CH), jnp.int32),
            pltpu.VMEM((2, CH), jnp.int32),
            pltpu.VMEM((CH, width), jnp.float32),
            pltpu.VMEM((CH, width), jnp.float32),
            pltpu.VMEM_SHARED((NP, width), jnp.float32),
            pltpu.SemaphoreType.DMA,
            pltpu.SemaphoreType.DMA,
            pltpu.SemaphoreType.DMA,
            pltpu.SemaphoreType.DMA,
            pltpu.SemaphoreType.DMA,
            pltpu.SemaphoreType.DMA,
        ],
    )
    def k(xws_hbm, src_hbm, dst_hbm, zero_hbm, out_hbm,
          sr0, sr1, dr0, dr1, rows0, rows1, acc,
          is0, is1, gs0, gs1, ss0, ss1):
        cid = lax.axis_index("c")
        sid = lax.axis_index("s")
        base = sid * RPT
        start = jnp.where(cid == 0, sid * KJ0, NS * KJ0 + sid * KJ1)
        npairs = jnp.where(cid == 0, KJ0 // 2, KJ1 // 2)

        def fetch_pair(i, sr, dr, sem):
            pltpu.async_copy(src_hbm.at[pl.ds(start + 2 * i, 2)], sr, sem)
            pltpu.async_copy(dst_hbm.at[pl.ds(start + 2 * i, 2)], dr, sem)

        def fetch_wait(sr, dr, sem):
            pltpu.make_async_copy(src_hbm.at[pl.ds(0, 2)], sr, sem).wait()
            pltpu.make_async_copy(dst_hbm.at[pl.ds(0, 2)], dr, sem).wait()

        def gather_start(sr, r, buf, sem):
            pltpu.async_copy(xws_hbm.at[sr.at[r]], buf, sem)

        def gather_wait(sr, buf, sem):
            pltpu.make_async_copy(xws_hbm.at[sr.at[0]], buf, sem).wait()

        def scatter_start(dr, r, buf, sem):
            pltpu.async_copy(buf, acc.at[dr.at[r]], sem, add=True)

        def scatter_wait(dr, buf, sem):
            pltpu.make_async_copy(buf, acc.at[dr.at[0]], sem).wait()

        fetch_pair(0, sr0, dr0, is0)
        pltpu.sync_copy(zero_hbm.at[pl.ds(base, RPT)], acc.at[pl.ds(base, RPT)])
        plsc.subcore_barrier()
        fetch_wait(sr0, dr0, is0)
        gather_start(sr0, 0, rows0, gs0)

        def pair_body(i, cur, nxt):
            (srX, drX, _), (srY, drY, isY) = cur, nxt
            gather_wait(srX, rows0, gs0)

            @pl.when(i > 0)
            def _():
                scatter_wait(drY, rows1, ss1)

            @pl.when(i + 1 < npairs)
            def _():
                fetch_pair(i + 1, srY, drY, isY)

            gather_start(srX, 1, rows1, gs1)
            scatter_start(drX, 0, rows0, ss0)
            gather_wait(srX, rows1, gs1)
            scatter_wait(drX, rows0, ss0)

            @pl.when(i + 1 < npairs)
            def _():
                fetch_wait(srY, drY, isY)
                gather_start(srY, 0, rows0, gs0)

            scatter_start(drX, 1, rows1, ss1)

        def body(t, carry):
            pair_body(2 * t, (sr0, dr0, is0), (sr1, dr1, is1))
            pair_body(2 * t + 1, (sr1, dr1, is1), (sr0, dr0, is0))
            return carry

        lax.fori_loop(0, npairs // 2, body, 0)
        scatter_wait(dr1, rows1, ss1)
        plsc.subcore_barrier()
        pltpu.sync_copy(acc.at[pl.ds(base, RPT)],
                        out_hbm.at[cid].at[pl.ds(base, RPT)])

    return k(xws, srcidx, dstidx, zeros)


def _sc_degree(dstidx, ones_rows, zerosw):
    mesh = plsc.VectorSubcoreMesh(core_axis_name="c", subcore_axis_name="s")

    def scatter_start(acc, dst_v, j, buf, sem):
        pltpu.async_copy(buf, acc.at[dst_v.at[j]], sem, add=True)

    def scatter_wait(acc, dst_v, buf, sem):
        pltpu.make_async_copy(buf, acc.at[dst_v.at[0]], sem).wait()

    @functools.partial(
        pl.kernel,
        out_type=jax.ShapeDtypeStruct((NC, NP, D), jnp.float32),
        mesh=mesh,
        scratch_types=[
            pltpu.VMEM((1, CH), jnp.int32),
            pltpu.VMEM((1, CH), jnp.int32),
            pltpu.VMEM((CH, D), jnp.float32),
            pltpu.VMEM_SHARED((NP, D), jnp.float32),
            pltpu.SemaphoreType.DMA,
            pltpu.SemaphoreType.DMA,
            pltpu.SemaphoreType.DMA,
            pltpu.SemaphoreType.DMA,
        ],
    )
    def k(dst_hbm, ones_hbm, zero_hbm, out_hbm, r0, r1, ones_v, acc,
          is0, is1, ss0, ss1):
        cid = lax.axis_index("c")
        sid = lax.axis_index("s")
        wid = cid * NS + sid
        base = sid * RPT

        def fetch(j, r, sem):
            pltpu.async_copy(dst_hbm.at[pl.ds(wid * KJ + j, 1)], r, sem)

        def fetch_wait(r, sem):
            pltpu.make_async_copy(dst_hbm.at[pl.ds(0, 1)], r, sem).wait()

        def scatter_start(r, sem):
            pltpu.async_copy(ones_v, acc.at[r.at[0]], sem, add=True)

        def scatter_wait(r, sem):
            pltpu.make_async_copy(ones_v, acc.at[r.at[0]], sem).wait()

        fetch(0, r0, is0)
        fetch(1, r1, is1)
        pltpu.sync_copy(ones_hbm, ones_v)
        pltpu.sync_copy(zero_hbm.at[pl.ds(base, RPT)], acc.at[pl.ds(base, RPT)])
        plsc.subcore_barrier()
        fetch_wait(r0, is0)
        scatter_start(r0, ss0)

        def body(t, carry):
            fetch_wait(r1, is1)
            scatter_start(r1, ss1)
            scatter_wait(r0, ss0)
            fetch(2 * t + 2, r0, is0)
            scatter_wait(r1, ss1)
            fetch(2 * t + 3, r1, is1)
            fetch_wait(r0, is0)
            scatter_start(r0, ss0)
            return carry

        lax.fori_loop(0, KJ // 2 - 1, body, 0)
        fetch_wait(r1, is1)
        scatter_start(r1, ss1)
        scatter_wait(r0, ss0)
        scatter_wait(r1, ss1)
        plsc.subcore_barrier()
        pltpu.sync_copy(acc.at[pl.ds(base, RPT)],
                        out_hbm.at[cid].at[pl.ds(base, RPT)])

    return k(dstidx, ones_rows, zerosw)


def _tc_prep(deg, x, W0):
    def body(deg_ref, x_ref, w_ref, dinv_ref, xws_ref):
        d = deg_ref[0] + deg_ref[1]
        dcol = d[:, 0:1] + 1.0
        dinv = lax.rsqrt(dcol)
        dinvb = jnp.broadcast_to(dinv, (128, D))
        xw = jnp.dot(x_ref[...], w_ref[...], preferred_element_type=jnp.float32)
        dinv_ref[...] = dinvb
        xws_ref[...] = xw * dinvb

    return pl.pallas_call(
        body,
        grid=(NB,),
        in_specs=[
            pl.BlockSpec((NC, 128, D), lambda i: (0, i, 0)),
            pl.BlockSpec((128, D), lambda i: (i, 0)),
            pl.BlockSpec((D, D), lambda i: (0, 0)),
        ],
        out_specs=[
            pl.BlockSpec((128, D), lambda i: (i, 0)),
            pl.BlockSpec((128, D), lambda i: (i, 0)),
        ],
        out_shape=[
            jax.ShapeDtypeStruct((NP, D), jnp.float32),
            jax.ShapeDtypeStruct((NP, D), jnp.float32),
        ],
    )(deg, x, W0)


def _tc_post(acc, xws, dinv, b):
    def body(acc_ref, xws_ref, dinv_ref, b_ref, agg_ref, st_ref):
        i = pl.program_id(0)
        s = acc_ref[0] + acc_ref[1] + xws_ref[...]
        agg = dinv_ref[...] * s + b_ref[...]
        agg_ref[...] = agg
        row = i * 128 + lax.broadcasted_iota(jnp.int32, (128, 1), 0)
        aggm = jnp.where(row < N, agg, 0.0)

        @pl.when(i == 0)
        def _():
            st_ref[...] = jnp.zeros_like(st_ref)

        st_ref[0:1, :] += jnp.sum(aggm, axis=0, keepdims=True)
        st_ref[1:2, :] += jnp.sum(aggm * aggm, axis=0, keepdims=True)

    return pl.pallas_call(
        body,
        grid=(NB,),
        in_specs=[
            pl.BlockSpec((NC, 128, D), lambda i: (0, i, 0)),
            pl.BlockSpec((128, D), lambda i: (i, 0)),
            pl.BlockSpec((128, D), lambda i: (i, 0)),
            pl.BlockSpec((1, D), lambda i: (0, 0)),
        ],
        out_specs=[
            pl.BlockSpec((128, D), lambda i: (i, 0)),
            pl.BlockSpec((2, D), lambda i: (0, 0)),
        ],
        out_shape=[
            jax.ShapeDtypeStruct((NP, D), jnp.float32),
            jax.ShapeDtypeStruct((2, D), jnp.float32),
        ],
    )(acc, xws, dinv, b)


def _bn_relu(agg, st, g, be):
    mu = st[0:1, :] * (1.0 / N)
    ex2 = st[1:2, :] * (1.0 / N)
    var = ex2 - mu * mu
    scale = g * lax.rsqrt(var + 1e-5)
    return jnp.maximum((agg - mu) * scale + be, 0.0)


def _tc_mm(agg, stats, g, be, W, dinv):
    def body(agg_ref, st_ref, g_ref, be_ref, w_ref, dinv_ref, xws_ref):
        h = _bn_relu(agg_ref[...], st_ref[...], g_ref[...], be_ref[...])
        xw = jnp.dot(h, w_ref[...], preferred_element_type=jnp.float32)
        xws_ref[...] = xw * dinv_ref[...]

    return pl.pallas_call(
        body,
        grid=(NB,),
        in_specs=[
            pl.BlockSpec((128, D), lambda i: (i, 0)),
            pl.BlockSpec((2, D), lambda i: (0, 0)),
            pl.BlockSpec((1, D), lambda i: (0, 0)),
            pl.BlockSpec((1, D), lambda i: (0, 0)),
            pl.BlockSpec((D, D), lambda i: (0, 0)),
            pl.BlockSpec((128, D), lambda i: (i, 0)),
        ],
        out_specs=pl.BlockSpec((128, D), lambda i: (i, 0)),
        out_shape=jax.ShapeDtypeStruct((NP, D), jnp.float32),
    )(agg, stats, g, be, W, dinv)


def _tc_final(agg, stats, g, be, batchb, linW, linb):
    def body(agg_ref, st_ref, g_ref, be_ref, bb_ref, lw_ref, lb_ref,
             out_ref, sums, cnts):
        i = pl.program_id(0)
        h = _bn_relu(agg_ref[...], st_ref[...], g_ref[...], be_ref[...])
        gid = lax.broadcasted_iota(jnp.int32, (128, 128), 1)
        p = (bb_ref[...] == gid).astype(jnp.float32)
        dn = (((0,), (0,)), ((), ()))
        ps = lax.dot_general(p, h, dn, preferred_element_type=jnp.float32)
        pc = lax.dot_general(p, jnp.ones((128, 1), jnp.float32), dn,
                             preferred_element_type=jnp.float32)

        @pl.when(i == 0)
        def _():
            sums[...] = jnp.zeros_like(sums)
            cnts[...] = jnp.zeros_like(cnts)

        sums[...] += ps
        cnts[...] += pc

        @pl.when(i == NB - 1)
        def _():
            pooled = sums[...] / jnp.maximum(cnts[...], 1.0)
            o = jnp.dot(pooled, lw_ref[...],
                        preferred_element_type=jnp.float32) + lb_ref[...]
            out_ref[...] = o[:NG, :]

    return pl.pallas_call(
        body,
        grid=(NB,),
        in_specs=[
            pl.BlockSpec((128, D), lambda i: (i, 0)),
            pl.BlockSpec((2, D), lambda i: (0, 0)),
            pl.BlockSpec((1, D), lambda i: (0, 0)),
            pl.BlockSpec((1, D), lambda i: (0, 0)),
            pl.BlockSpec((128, 128), lambda i: (i, 0)),
            pl.BlockSpec((D, DOUT), lambda i: (0, 0)),
            pl.BlockSpec((1, DOUT), lambda i: (0, 0)),
        ],
        out_specs=pl.BlockSpec((NG, DOUT), lambda i: (0, 0)),
        out_shape=jax.ShapeDtypeStruct((NG, DOUT), jnp.float32),
        scratch_shapes=[
            pltpu.VMEM((128, 128), jnp.float32),
            pltpu.VMEM((128, 1), jnp.float32),
        ],
    )(agg, stats, g, be, batchb, linW, linb)


def kernel(x, edge_index, batch, W0, b0, g0, be0, W1, b1, g1, be1,
           W2, b2, g2, be2, linW, linb):
    f32 = jnp.float32
    xp = jnp.zeros((NP, D), f32).at[:N].set(x)
    pad = jnp.full((EP - E,), N, jnp.int32)
    srcp = jnp.concatenate([edge_index[0].astype(jnp.int32), pad]
                           ).reshape(TCH, CH)
    dstp = jnp.concatenate([edge_index[1].astype(jnp.int32), pad]
                           ).reshape(TCH, CH)
    batchp = jnp.concatenate([batch.astype(jnp.int32),
                              jnp.full((NP - N,), NG, jnp.int32)])
    batchb = jnp.broadcast_to(batchp[:, None], (NP, 128))
    zerosD = jnp.zeros((NP, D), f32)
    ones128 = jnp.ones((CH, D), f32)

    deg = _sc_degree(dstp, ones128, zerosD)
    dinv, xws = _tc_prep(deg, xp, W0)

    layers = ((b0, g0, be0, W1), (b1, g1, be1, W2), (b2, g2, be2, None))
    agg = stats = None
    for b, g, be, Wnext in layers:
        acc = _sc_edge_scatter(xws, srcp, dstp, zerosD, D)
        agg, stats = _tc_post(acc, xws, dinv, b.reshape(1, D))
        if Wnext is not None:
            xws = _tc_mm(agg, stats, g.reshape(1, D), be.reshape(1, D),
                         Wnext, dinv)

    return _tc_final(agg, stats, g2.reshape(1, D), be2.reshape(1, D),
                     batchb, linW, linb.reshape(1, DOUT))


# speedup vs baseline: 7.7689x; 1.0204x over previous
import functools

import jax
import jax.numpy as jnp
from jax import lax
from jax.experimental import pallas as pl
from jax.experimental.pallas import tpu as pltpu
from jax.experimental.pallas import tpu_sc as plsc

N = 10000
NP = 10112
NB = NP // 128
E = 320000
NC = 2
NS = 16
NT = NC * NS
CH = 128
KJ = 80
PAIRS = KJ // 2
EP = NT * KJ * CH
TCH = EP // CH
KJ0 = 148
KJ1 = 12
RPT = NP // NS
D = 128
DOUT = 64
NG = 64


def _sc_edge_scatter(xws, srcidx, dstidx, zeros, width):
    mesh = plsc.VectorSubcoreMesh(core_axis_name="c", subcore_axis_name="s")

    @functools.partial(
        pl.kernel,
        out_type=jax.ShapeDtypeStruct((NC, NP, width), jnp.float32),
        mesh=mesh,
        scratch_types=[
            pltpu.VMEM((2, CH), jnp.int32),
            pltpu.VMEM((2, CH), jnp.int32),
            pltpu.VMEM((2, CH), jnp.int32),
            pltpu.VMEM((2, CH), jnp.int32),
            pltpu.VMEM((CH, width), jnp.float32),
            pltpu.VMEM((CH, width), jnp.float32),
            pltpu.VMEM_SHARED((NP, width), jnp.float32),
            pltpu.SemaphoreType.DMA,
            pltpu.SemaphoreType.DMA,
            pltpu.SemaphoreType.DMA,
            pltpu.SemaphoreType.DMA,
            pltpu.SemaphoreType.DMA,
            pltpu.SemaphoreType.DMA,
        ],
    )
    def k(xws_hbm, src_hbm, dst_hbm, zero_hbm, out_hbm,
          sr0, sr1, dr0, dr1, rows0, rows1, acc,
          is0, is1, gs0, gs1, ss0, ss1):
        cid = lax.axis_index("c")
        sid = lax.axis_index("s")
        base = sid * RPT
        start = jnp.where(cid == 0, sid * KJ0, NS * KJ0 + sid * KJ1)
        npairs = jnp.where(cid == 0, KJ0 // 2, KJ1 // 2)

        def fetch_pair(i, sr, dr, sem):
            pltpu.async_copy(src_hbm.at[pl.ds(start + 2 * i, 2)], sr, sem)
            pltpu.async_copy(dst_hbm.at[pl.ds(start + 2 * i, 2)], dr, sem)

        def fetch_wait(sr, dr, sem):
            pltpu.make_async_copy(src_hbm.at[pl.ds(0, 2)], sr, sem).wait()
            pltpu.make_async_copy(dst_hbm.at[pl.ds(0, 2)], dr, sem).wait()

        def gather_start(sr, r, buf, sem):
            pltpu.async_copy(xws_hbm.at[sr.at[r]], buf, sem)

        def gather_wait(sr, buf, sem):
            pltpu.make_async_copy(xws_hbm.at[sr.at[0]], buf, sem).wait()

        def scatter_start(dr, r, buf, sem):
            pltpu.async_copy(buf, acc.at[dr.at[r]], sem, add=True)

        def scatter_wait(dr, buf, sem):
            pltpu.make_async_copy(buf, acc.at[dr.at[0]], sem).wait()

        @pl.when(npairs > 0)
        def _():
            fetch_pair(0, sr0, dr0, is0)

        pltpu.sync_copy(zero_hbm.at[pl.ds(base, RPT)], acc.at[pl.ds(base, RPT)])
        plsc.subcore_barrier()

        @pl.when(npairs > 0)
        def _():
            fetch_wait(sr0, dr0, is0)
            gather_start(sr0, 0, rows0, gs0)

        def pair_body(i, cur, nxt):
            (srX, drX, _), (srY, drY, isY) = cur, nxt
            gather_wait(srX, rows0, gs0)

            @pl.when(i > 0)
            def _():
                scatter_wait(drY, rows1, ss1)

            @pl.when(i + 1 < npairs)
            def _():
                fetch_pair(i + 1, srY, drY, isY)

            gather_start(srX, 1, rows1, gs1)
            scatter_start(drX, 0, rows0, ss0)
            gather_wait(srX, rows1, gs1)
            scatter_wait(drX, rows0, ss0)

            @pl.when(i + 1 < npairs)
            def _():
                fetch_wait(srY, drY, isY)
                gather_start(srY, 0, rows0, gs0)

            scatter_start(drX, 1, rows1, ss1)

        def body(t, carry):
            pair_body(2 * t, (sr0, dr0, is0), (sr1, dr1, is1))
            pair_body(2 * t + 1, (sr1, dr1, is1), (sr0, dr0, is0))
            return carry

        lax.fori_loop(0, npairs // 2, body, 0)

        @pl.when(npairs > 0)
        def _():
            scatter_wait(dr1, rows1, ss1)

        plsc.subcore_barrier()
        pltpu.sync_copy(acc.at[pl.ds(base, RPT)],
                        out_hbm.at[cid].at[pl.ds(base, RPT)])

    return k(xws, srcidx, dstidx, zeros)


def _sc_degree(dstidx, ones_rows, zerosw):
    mesh = plsc.VectorSubcoreMesh(core_axis_name="c", subcore_axis_name="s")

    def scatter_start(acc, dst_v, j, buf, sem):
        pltpu.async_copy(buf, acc.at[dst_v.at[j]], sem, add=True)

    def scatter_wait(acc, dst_v, buf, sem):
        pltpu.make_async_copy(buf, acc.at[dst_v.at[0]], sem).wait()

    @functools.partial(
        pl.kernel,
        out_type=jax.ShapeDtypeStruct((NC, NP, D), jnp.float32),
        mesh=mesh,
        scratch_types=[
            pltpu.VMEM((1, CH), jnp.int32),
            pltpu.VMEM((1, CH), jnp.int32),
            pltpu.VMEM((CH, D), jnp.float32),
            pltpu.VMEM_SHARED((NP, D), jnp.float32),
            pltpu.SemaphoreType.DMA,
            pltpu.SemaphoreType.DMA,
            pltpu.SemaphoreType.DMA,
            pltpu.SemaphoreType.DMA,
        ],
    )
    def k(dst_hbm, ones_hbm, zero_hbm, out_hbm, r0, r1, ones_v, acc,
          is0, is1, ss0, ss1):
        cid = lax.axis_index("c")
        sid = lax.axis_index("s")
        wid = cid * NS + sid
        base = sid * RPT

        def fetch(j, r, sem):
            pltpu.async_copy(dst_hbm.at[pl.ds(wid * KJ + j, 1)], r, sem)

        def fetch_wait(r, sem):
            pltpu.make_async_copy(dst_hbm.at[pl.ds(0, 1)], r, sem).wait()

        def scatter_start(r, sem):
            pltpu.async_copy(ones_v, acc.at[r.at[0]], sem, add=True)

        def scatter_wait(r, sem):
            pltpu.make_async_copy(ones_v, acc.at[r.at[0]], sem).wait()

        fetch(0, r0, is0)
        fetch(1, r1, is1)
        pltpu.sync_copy(ones_hbm, ones_v)
        pltpu.sync_copy(zero_hbm.at[pl.ds(base, RPT)], acc.at[pl.ds(base, RPT)])
        plsc.subcore_barrier()
        fetch_wait(r0, is0)
        scatter_start(r0, ss0)

        def body(t, carry):
            fetch_wait(r1, is1)
            scatter_start(r1, ss1)
            scatter_wait(r0, ss0)
            fetch(2 * t + 2, r0, is0)
            scatter_wait(r1, ss1)
            fetch(2 * t + 3, r1, is1)
            fetch_wait(r0, is0)
            scatter_start(r0, ss0)
            return carry

        lax.fori_loop(0, KJ // 2 - 1, body, 0)
        fetch_wait(r1, is1)
        scatter_start(r1, ss1)
        scatter_wait(r0, ss0)
        scatter_wait(r1, ss1)
        plsc.subcore_barrier()
        pltpu.sync_copy(acc.at[pl.ds(base, RPT)],
                        out_hbm.at[cid].at[pl.ds(base, RPT)])

    return k(dstidx, ones_rows, zerosw)


def _tc_prep(deg, x, W0):
    def body(deg_ref, x_ref, w_ref, dinv_ref, xws_ref):
        d = deg_ref[0] + deg_ref[1]
        dcol = d[:, 0:1] + 1.0
        dinv = lax.rsqrt(dcol)
        dinvb = jnp.broadcast_to(dinv, (128, D))
        xw = jnp.dot(x_ref[...], w_ref[...], preferred_element_type=jnp.float32)
        dinv_ref[...] = dinvb
        xws_ref[...] = xw * dinvb

    return pl.pallas_call(
        body,
        grid=(NB,),
        in_specs=[
            pl.BlockSpec((NC, 128, D), lambda i: (0, i, 0)),
            pl.BlockSpec((128, D), lambda i: (i, 0)),
            pl.BlockSpec((D, D), lambda i: (0, 0)),
        ],
        out_specs=[
            pl.BlockSpec((128, D), lambda i: (i, 0)),
            pl.BlockSpec((128, D), lambda i: (i, 0)),
        ],
        out_shape=[
            jax.ShapeDtypeStruct((NP, D), jnp.float32),
            jax.ShapeDtypeStruct((NP, D), jnp.float32),
        ],
    )(deg, x, W0)


def _tc_post(acc, xws, dinv, b):
    def body(acc_ref, xws_ref, dinv_ref, b_ref, agg_ref, st_ref):
        i = pl.program_id(0)
        s = acc_ref[0] + acc_ref[1] + xws_ref[...]
        agg = dinv_ref[...] * s + b_ref[...]
        agg_ref[...] = agg
        row = i * 128 + lax.broadcasted_iota(jnp.int32, (128, 1), 0)
        aggm = jnp.where(row < N, agg, 0.0)

        @pl.when(i == 0)
        def _():
            st_ref[...] = jnp.zeros_like(st_ref)

        st_ref[0:1, :] += jnp.sum(aggm, axis=0, keepdims=True)
        st_ref[1:2, :] += jnp.sum(aggm * aggm, axis=0, keepdims=True)

    return pl.pallas_call(
        body,
        grid=(NB,),
        in_specs=[
            pl.BlockSpec((NC, 128, D), lambda i: (0, i, 0)),
            pl.BlockSpec((128, D), lambda i: (i, 0)),
            pl.BlockSpec((128, D), lambda i: (i, 0)),
            pl.BlockSpec((1, D), lambda i: (0, 0)),
        ],
        out_specs=[
            pl.BlockSpec((128, D), lambda i: (i, 0)),
            pl.BlockSpec((2, D), lambda i: (0, 0)),
        ],
        out_shape=[
            jax.ShapeDtypeStruct((NP, D), jnp.float32),
            jax.ShapeDtypeStruct((2, D), jnp.float32),
        ],
    )(acc, xws, dinv, b)


def _bn_relu(agg, st, g, be):
    mu = st[0:1, :] * (1.0 / N)
    ex2 = st[1:2, :] * (1.0 / N)
    var = ex2 - mu * mu
    scale = g * lax.rsqrt(var + 1e-5)
    return jnp.maximum((agg - mu) * scale + be, 0.0)


def _tc_mm(agg, stats, g, be, W, dinv):
    def body(agg_ref, st_ref, g_ref, be_ref, w_ref, dinv_ref, xws_ref):
        h = _bn_relu(agg_ref[...], st_ref[...], g_ref[...], be_ref[...])
        xw = jnp.dot(h, w_ref[...], preferred_element_type=jnp.float32)
        xws_ref[...] = xw * dinv_ref[...]

    return pl.pallas_call(
        body,
        grid=(NB,),
        in_specs=[
            pl.BlockSpec((128, D), lambda i: (i, 0)),
            pl.BlockSpec((2, D), lambda i: (0, 0)),
            pl.BlockSpec((1, D), lambda i: (0, 0)),
            pl.BlockSpec((1, D), lambda i: (0, 0)),
            pl.BlockSpec((D, D), lambda i: (0, 0)),
            pl.BlockSpec((128, D), lambda i: (i, 0)),
        ],
        out_specs=pl.BlockSpec((128, D), lambda i: (i, 0)),
        out_shape=jax.ShapeDtypeStruct((NP, D), jnp.float32),
    )(agg, stats, g, be, W, dinv)


def _tc_final(agg, stats, g, be, batchb, linW, linb):
    def body(agg_ref, st_ref, g_ref, be_ref, bb_ref, lw_ref, lb_ref,
             out_ref, sums, cnts):
        i = pl.program_id(0)
        h = _bn_relu(agg_ref[...], st_ref[...], g_ref[...], be_ref[...])
        gid = lax.broadcasted_iota(jnp.int32, (128, 128), 1)
        p = (bb_ref[...] == gid).astype(jnp.float32)
        dn = (((0,), (0,)), ((), ()))
        ps = lax.dot_general(p, h, dn, preferred_element_type=jnp.float32)
        pc = lax.dot_general(p, jnp.ones((128, 1), jnp.float32), dn,
                             preferred_element_type=jnp.float32)

        @pl.when(i == 0)
        def _():
            sums[...] = jnp.zeros_like(sums)
            cnts[...] = jnp.zeros_like(cnts)

        sums[...] += ps
        cnts[...] += pc

        @pl.when(i == NB - 1)
        def _():
            pooled = sums[...] / jnp.maximum(cnts[...], 1.0)
            o = jnp.dot(pooled, lw_ref[...],
                        preferred_element_type=jnp.float32) + lb_ref[...]
            out_ref[...] = o[:NG, :]

    return pl.pallas_call(
        body,
        grid=(NB,),
        in_specs=[
            pl.BlockSpec((128, D), lambda i: (i, 0)),
            pl.BlockSpec((2, D), lambda i: (0, 0)),
            pl.BlockSpec((1, D), lambda i: (0, 0)),
            pl.BlockSpec((1, D), lambda i: (0, 0)),
            pl.BlockSpec((128, 128), lambda i: (i, 0)),
            pl.BlockSpec((D, DOUT), lambda i: (0, 0)),
            pl.BlockSpec((1, DOUT), lambda i: (0, 0)),
        ],
        out_specs=pl.BlockSpec((NG, DOUT), lambda i: (0, 0)),
        out_shape=jax.ShapeDtypeStruct((NG, DOUT), jnp.float32),
        scratch_shapes=[
            pltpu.VMEM((128, 128), jnp.float32),
            pltpu.VMEM((128, 1), jnp.float32),
        ],
    )(agg, stats, g, be, batchb, linW, linb)


def kernel(x, edge_index, batch, W0, b0, g0, be0, W1, b1, g1, be1,
           W2, b2, g2, be2, linW, linb):
    f32 = jnp.float32
    xp = jnp.zeros((NP, D), f32).at[:N].set(x)
    pad = jnp.full((EP - E,), N, jnp.int32)
    srcp = jnp.concatenate([edge_index[0].astype(jnp.int32), pad]
                           ).reshape(TCH, CH)
    dstp = jnp.concatenate([edge_index[1].astype(jnp.int32), pad]
                           ).reshape(TCH, CH)
    batchp = jnp.concatenate([batch.astype(jnp.int32),
                              jnp.full((NP - N,), NG, jnp.int32)])
    batchb = jnp.broadcast_to(batchp[:, None], (NP, 128))
    zerosD = jnp.zeros((NP, D), f32)
    ones128 = jnp.ones((CH, D), f32)

    deg = _sc_degree(dstp, ones128, zerosD)
    dinv, xws = _tc_prep(deg, xp, W0)

    layers = ((b0, g0, be0, W1), (b1, g1, be1, W2), (b2, g2, be2, None))
    agg = stats = None
    for b, g, be, Wnext in layers:
        acc = _sc_edge_scatter(xws, srcp, dstp, zerosD, D)
        agg, stats = _tc_post(acc, xws, dinv, b.reshape(1, D))
        if Wnext is not None:
            xws = _tc_mm(agg, stats, g.reshape(1, D), be.reshape(1, D),
                         Wnext, dinv)

    return _tc_final(agg, stats, g2.reshape(1, D), be2.reshape(1, D),
                     batchb, linW, linb.reshape(1, DOUT))
